# baseline (speedup 1.0000x reference)
import functools

import jax
import jax.numpy as jnp
from jax import lax
from jax.experimental import pallas as pl
from jax.experimental.pallas import tpu as pltpu
from jax.experimental.pallas import tpu_sc as plsc

_N = 10000
_K = 16
_RB = 200
_GRID = _N // _RB
_RBT = 400
_GRIDT = _N // _RBT
_E = _N * _K
_NP = 10112
_NCH = _NP // 128
_CW = 128
_F = 64
_GF = 128


def _topk_uv_body(posb_ref, post_ref, wsum_ref, wb_ref, idx_ref, u_ref, v_ref,
                  d_scr):
    posb = posb_ref[...]
    post = post_ref[...]
    inf = jnp.float32(jnp.inf)
    big = jnp.int32(2 ** 30)
    coli = lax.broadcasted_iota(jnp.int32, (1, _NP), 1)
    sqt = jnp.sum(post * post, axis=0, keepdims=True)
    sqt = jnp.where(coli < _N, sqt, inf)
    sqb = jnp.sum(posb * posb, axis=1, keepdims=True)
    g = jnp.dot(posb, post, preferred_element_type=jnp.float32)
    d_scr[...] = sqb + sqt - 2.0 * g

    lanei = lax.broadcasted_iota(jnp.int32, (_RBT, 128), 1)
    a1 = jnp.full((_RBT, 128), inf)
    a2, a3, a4 = a1, a1, a1
    c1 = jnp.zeros((_RBT, 128), jnp.int32)
    c2, c3, c4 = c1, c1, c1
    for c in range(_NCH):
        dc = d_scr[:, c * 128:(c + 1) * 128]
        cols = lanei + (c * 128)
        lt1 = dc < a1
        lt2 = dc < a2
        lt3 = dc < a3
        lt4 = dc < a4
        a4 = jnp.where(lt4, jnp.where(lt3, a3, dc), a4)
        c4 = jnp.where(lt4, jnp.where(lt3, c3, cols), c4)
        a3 = jnp.where(lt3, jnp.where(lt2, a2, dc), a3)
        c3 = jnp.where(lt3, jnp.where(lt2, c2, cols), c3)
        a2 = jnp.where(lt2, jnp.where(lt1, a1, dc), a2)
        c2 = jnp.where(lt2, jnp.where(lt1, c1, cols), c2)
        a1 = jnp.where(lt1, dc, a1)
        c1 = jnp.where(lt1, cols, c1)

    outcols = []
    for k in range(_K):
        m = jnp.min(a1, axis=1, keepdims=True)
        lane = jnp.min(jnp.where(a1 <= m, lanei, big), axis=1, keepdims=True)
        islane = lanei == lane
        col = jnp.min(jnp.where(islane, c1, big), axis=1, keepdims=True)
        outcols.append(col)
        a1 = jnp.where(islane, a2, a1)
        c1 = jnp.where(islane, c2, c1)
        a2 = jnp.where(islane, a3, a2)
        c2 = jnp.where(islane, c3, c2)
        a3 = jnp.where(islane, a4, a3)
        c3 = jnp.where(islane, c4, c3)
        a4 = jnp.where(islane, inf, a4)
    idx_fast = jnp.concatenate(outcols, axis=1)

    bad = jnp.any(a1 == inf)

    @pl.when(jnp.logical_not(bad))
    def _fast():
        idx_ref[...] = idx_fast

    @pl.when(bad)
    def _slow():
        iota = lax.broadcasted_iota(jnp.int32, (_RBT, _NP), 1)
        slot = lax.broadcasted_iota(jnp.int32, (_RBT, _K), 1)

        def step(k, acc):
            d = d_scr[...]
            mn = jnp.min(d, axis=1, keepdims=True)
            am = jnp.min(jnp.where(d <= mn, iota, big), axis=1, keepdims=True)
            d_scr[...] = jnp.where(iota == am, inf, d)
            return jnp.where(slot == k, am, acc)

        idx_ref[...] = lax.fori_loop(0, _K, step, jnp.zeros((_RBT, _K),
                                                            jnp.int32))

    u_ref[...] = jnp.dot(posb, wsum_ref[...], preferred_element_type=jnp.float32)
    v_ref[...] = jnp.dot(posb, wb_ref[...], preferred_element_type=jnp.float32)


def _conv1_body(gu_ref, v_ref, posb_ref, b1_ref, b2_ref, w2_ref, w3a_ref,
                w3b_ref, c_ref, e_ref):
    gu = gu_ref[:, :, : _F]
    pre = gu - v_ref[...][:, None, :] + b1_ref[...][None, :, :]
    h = jnp.maximum(pre, 0.0).reshape(_RB * _K, _F)
    h2 = jnp.dot(h, w2_ref[...], preferred_element_type=jnp.float32)
    x = jnp.max(h2.reshape(_RB, _K, _F), axis=1) + b2_ref[...]
    pw = jnp.dot(posb_ref[...], w3b_ref[...], preferred_element_type=jnp.float32)
    c_ref[...] = jnp.dot(x, w3a_ref[...], preferred_element_type=jnp.float32) + pw
    e_ref[...] = pw[:, : _F]


def _conv2_body(gc_ref, e_ref, b3_ref, w4_ref, b4_ref, w5_ref, b5_ref, w6_ref,
                b6_ref, out_ref, acc_scr):
    i = pl.program_id(0)
    h1 = jnp.maximum(
        gc_ref[:, :, : _F] - e_ref[...][:, None, :] + b3_ref[...][None, :, :],
        0.0)
    h1 = h1.reshape(_RB * _K, _F)
    h2 = jnp.maximum(
        jnp.dot(h1, w4_ref[...], preferred_element_type=jnp.float32)
        + b4_ref[...], 0.0)
    h3 = jnp.dot(h2.astype(jnp.bfloat16), w5_ref[...],
                 preferred_element_type=jnp.float32)
    gblk = jnp.max(h3, axis=0, keepdims=True)

    @pl.when(i == 0)
    def _init():
        acc_scr[...] = gblk

    @pl.when(i > 0)
    def _acc():
        acc_scr[...] = jnp.maximum(acc_scr[...], gblk)

    @pl.when(i == _GRID - 1)
    def _fin():
        gfin = acc_scr[...] + b5_ref[...]
        out_ref[...] = jnp.maximum(
            jnp.dot(gfin, w6_ref[...], preferred_element_type=jnp.float32)
            + b6_ref[...], 0.0)


@functools.cache
def _gather_fn(ch):
    info = plsc.get_sparse_core_info()
    nc, ns = info.num_cores, info.num_subcores
    nw = nc * ns
    epad = nw * ch * _CW
    mesh = plsc.VectorSubcoreMesh(core_axis_name="c", subcore_axis_name="s")

    @functools.partial(
        pl.kernel,
        mesh=mesh,
        out_type=jax.ShapeDtypeStruct((epad, _GF), jnp.float32),
        scratch_types=[
            pltpu.VMEM((ch, _CW), jnp.int32),
            pltpu.VMEM((_CW, _GF), jnp.float32),
            pltpu.VMEM((_CW, _GF), jnp.float32),
            pltpu.SemaphoreType.DMA,
            pltpu.SemaphoreType.DMA,
        ],
    )
    def gather(table_hbm, idx_hbm, out_hbm, idx_v, rows_a, rows_b, sem_a,
               sem_b):
        wid = lax.axis_index("s") * nc + lax.axis_index("c")
        base = wid * ch * _CW
        pltpu.sync_copy(idx_hbm.at[wid], idx_v)
        pltpu.async_copy(table_hbm.at[idx_v.at[0]], rows_a, sem_a)

        def pair(p, carry):
            j = p * 2
            pltpu.async_copy(table_hbm.at[idx_v.at[j + 1]], rows_b, sem_b)
            pltpu.make_async_copy(table_hbm.at[idx_v.at[j]], rows_a,
                                  sem_a).wait()
            pltpu.sync_copy(rows_a, out_hbm.at[pl.ds(base + j * _CW, _CW)])

            @pl.when(p + 1 < ch // 2)
            def _next():
                pltpu.async_copy(table_hbm.at[idx_v.at[j + 2]], rows_a, sem_a)

            pltpu.make_async_copy(table_hbm.at[idx_v.at[j + 1]], rows_b,
                                  sem_b).wait()
            pltpu.sync_copy(rows_b, out_hbm.at[pl.ds(base + (j + 1) * _CW,
                                                     _CW)])
            return carry

        lax.fori_loop(0, ch // 2, pair, 0)

    return gather


def _sc_gather(table, idx3, ch):
    return _gather_fn(ch)(table, idx3)


def kernel(pos, batch, W1, b1, W2, b2, W3, b3, W4, b4, W5, b5, W6, b6):
    del batch
    info = plsc.get_sparse_core_info()
    nw = info.num_cores * info.num_subcores
    ch = -(-_E // (nw * _CW))
    ch += ch % 2
    epad = nw * ch * _CW

    post = jnp.pad(pos.T, ((0, 0), (0, _NP - _N)))
    wsum = jnp.pad(W1[:3] + W1[3:], ((0, 0), (0, _GF - _F)))
    wb = W1[3:]
    idx, u, v = pl.pallas_call(
        _topk_uv_body,
        grid=(_GRIDT,),
        in_specs=[
            pl.BlockSpec((_RBT, 3), lambda i: (i, 0)),
            pl.BlockSpec((3, _NP), lambda i: (0, 0)),
            pl.BlockSpec((3, _GF), lambda i: (0, 0)),
            pl.BlockSpec((3, _F), lambda i: (0, 0)),
        ],
        out_specs=[
            pl.BlockSpec((_RBT, _K), lambda i: (i, 0)),
            pl.BlockSpec((_RBT, _GF), lambda i: (i, 0)),
            pl.BlockSpec((_RBT, _F), lambda i: (i, 0)),
        ],
        out_shape=[
            jax.ShapeDtypeStruct((_N, _K), jnp.int32),
            jax.ShapeDtypeStruct((_N, _GF), jnp.float32),
            jax.ShapeDtypeStruct((_N, _F), jnp.float32),
        ],
        scratch_shapes=[pltpu.VMEM((_RBT, _NP), jnp.float32)],
    )(pos, post, wsum, wb)

    src = idx.reshape(-1)
    pad = (jnp.arange(epad - _E, dtype=jnp.int32) * 37) % _N
    src3 = jnp.concatenate([src, pad]).reshape(nw, ch, _CW)

    gu3 = _sc_gather(u, src3, ch).reshape(epad // _K, _K, _GF)

    b1r, b2r = b1.reshape(1, _F), b2.reshape(1, _F)
    w3a = jnp.pad(W3[:_F], ((0, 0), (0, _GF - _F)))
    w3b = jnp.pad(W3[_F:], ((0, 0), (0, _GF - _F)))
    c, e = pl.pallas_call(
        _conv1_body,
        grid=(_GRID,),
        in_specs=[
            pl.BlockSpec((_RB, _K, _GF), lambda i: (i, 0, 0)),
            pl.BlockSpec((_RB, _F), lambda i: (i, 0)),
            pl.BlockSpec((_RB, 3), lambda i: (i, 0)),
            pl.BlockSpec((1, _F), lambda i: (0, 0)),
            pl.BlockSpec((1, _F), lambda i: (0, 0)),
            pl.BlockSpec((_F, _F), lambda i: (0, 0)),
            pl.BlockSpec((_F, _GF), lambda i: (0, 0)),
            pl.BlockSpec((3, _GF), lambda i: (0, 0)),
        ],
        out_specs=[
            pl.BlockSpec((_RB, _GF), lambda i: (i, 0)),
            pl.BlockSpec((_RB, _F), lambda i: (i, 0)),
        ],
        out_shape=[
            jax.ShapeDtypeStruct((_N, _GF), jnp.float32),
            jax.ShapeDtypeStruct((_N, _F), jnp.float32),
        ],
    )(gu3, v, pos, b1r, b2r, W2, w3a, w3b)

    gc3 = _sc_gather(c, src3, ch).reshape(epad // _K, _K, _GF)

    out = pl.pallas_call(
        _conv2_body,
        grid=(_GRID,),
        in_specs=[
            pl.BlockSpec((_RB, _K, _GF), lambda i: (i, 0, 0)),
            pl.BlockSpec((_RB, _F), lambda i: (i, 0)),
            pl.BlockSpec((1, _F), lambda i: (0, 0)),
            pl.BlockSpec((_F, 128), lambda i: (0, 0)),
            pl.BlockSpec((1, 128), lambda i: (0, 0)),
            pl.BlockSpec((128, 1024), lambda i: (0, 0)),
            pl.BlockSpec((1, 1024), lambda i: (0, 0)),
            pl.BlockSpec((1024, 256), lambda i: (0, 0)),
            pl.BlockSpec((1, 256), lambda i: (0, 0)),
        ],
        out_specs=pl.BlockSpec((1, 256), lambda i: (0, 0)),
        out_shape=jax.ShapeDtypeStruct((1, 256), jnp.float32),
        scratch_shapes=[pltpu.VMEM((1, 1024), jnp.float32)],
    )(gc3, e, b3.reshape(1, _F), W4, b4.reshape(1, 128),
      W5.astype(jnp.bfloat16), b5.reshape(1, 1024), W6, b6.reshape(1, 256))
    return out


# speedup vs baseline: 1.3631x; 1.3631x over previous
import functools

import jax
import jax.numpy as jnp
from jax import lax
from jax.experimental import pallas as pl
from jax.experimental.pallas import tpu as pltpu
from jax.experimental.pallas import tpu_sc as plsc

_N = 10000
_K = 16
_RB = 200
_GRID = _N // _RB
_RBT = 400
_GRIDT = _N // _RBT
_E = _N * _K
_NP = 10112
_NCH = _NP // 128
_CW = 128
_F = 64
_GF = 128


def _topk_uv_body(posb_ref, post_ref, wsum_ref, wb_ref, idx_ref, u_ref, v_ref,
                  d_scr):
    posb = posb_ref[...]
    post = post_ref[...]
    inf = jnp.float32(jnp.inf)
    big = jnp.int32(2 ** 30)
    coli = lax.broadcasted_iota(jnp.int32, (1, _NP), 1)
    sqt = jnp.sum(post * post, axis=0, keepdims=True)
    sqt = jnp.where(coli < _N, sqt, inf)
    sqb = jnp.sum(posb * posb, axis=1, keepdims=True)
    g = jnp.dot(posb, post, preferred_element_type=jnp.float32)
    d_scr[...] = sqb + sqt - 2.0 * g

    lanei = lax.broadcasted_iota(jnp.int32, (_RBT, 128), 1)
    a1 = jnp.full((_RBT, 128), inf)
    a2, a3, a4, a5 = a1, a1, a1, a1
    c1 = jnp.zeros((_RBT, 128), jnp.int32)
    c2, c3, c4, c5 = c1, c1, c1, c1
    for c in range(_NCH):
        dc = d_scr[:, c * 128:(c + 1) * 128]
        cols = lanei + (c * 128)
        lt1 = dc < a1
        lt2 = dc < a2
        lt3 = dc < a3
        lt4 = dc < a4
        lt5 = dc < a5
        a5 = jnp.where(lt5, jnp.where(lt4, a4, dc), a5)
        c5 = jnp.where(lt5, jnp.where(lt4, c4, cols), c5)
        a4 = jnp.where(lt4, jnp.where(lt3, a3, dc), a4)
        c4 = jnp.where(lt4, jnp.where(lt3, c3, cols), c4)
        a3 = jnp.where(lt3, jnp.where(lt2, a2, dc), a3)
        c3 = jnp.where(lt3, jnp.where(lt2, c2, cols), c3)
        a2 = jnp.where(lt2, jnp.where(lt1, a1, dc), a2)
        c2 = jnp.where(lt2, jnp.where(lt1, c1, cols), c2)
        a1 = jnp.where(lt1, dc, a1)
        c1 = jnp.where(lt1, cols, c1)

    outcols = []
    for k in range(_K):
        m = jnp.min(a1, axis=1, keepdims=True)
        lane = jnp.min(jnp.where(a1 <= m, lanei, big), axis=1, keepdims=True)
        islane = lanei == lane
        col = jnp.min(jnp.where(islane, c1, big), axis=1, keepdims=True)
        outcols.append(col)
        a1 = jnp.where(islane, a2, a1)
        c1 = jnp.where(islane, c2, c1)
        a2 = jnp.where(islane, a3, a2)
        c2 = jnp.where(islane, c3, c2)
        a3 = jnp.where(islane, a4, a3)
        c3 = jnp.where(islane, c4, c3)
        a4 = jnp.where(islane, a5, a4)
        c4 = jnp.where(islane, c5, c4)
        a5 = jnp.where(islane, inf, a5)
    idx_fast = jnp.concatenate(outcols, axis=1)

    bad = jnp.any(a1 == inf)

    @pl.when(jnp.logical_not(bad))
    def _fast():
        idx_ref[...] = idx_fast

    @pl.when(bad)
    def _slow():
        iota = lax.broadcasted_iota(jnp.int32, (_RBT, _NP), 1)
        slot = lax.broadcasted_iota(jnp.int32, (_RBT, _K), 1)

        def step(k, acc):
            d = d_scr[...]
            mn = jnp.min(d, axis=1, keepdims=True)
            am = jnp.min(jnp.where(d <= mn, iota, big), axis=1, keepdims=True)
            d_scr[...] = jnp.where(iota == am, inf, d)
            return jnp.where(slot == k, am, acc)

        idx_ref[...] = lax.fori_loop(0, _K, step, jnp.zeros((_RBT, _K),
                                                            jnp.int32))

    u_ref[...] = jnp.dot(posb, wsum_ref[...], preferred_element_type=jnp.float32)
    v_ref[...] = jnp.dot(posb, wb_ref[...], preferred_element_type=jnp.float32)


def _conv1_body(gu_ref, v_ref, posb_ref, b1_ref, b2_ref, w2_ref, w3a_ref,
                w3b_ref, c_ref, e_ref):
    gu = gu_ref[:, :, : _F]
    pre = gu - v_ref[...][:, None, :] + b1_ref[...][None, :, :]
    h = jnp.maximum(pre, 0.0).reshape(_RB * _K, _F)
    h2 = jnp.dot(h, w2_ref[...], preferred_element_type=jnp.float32)
    x = jnp.max(h2.reshape(_RB, _K, _F), axis=1) + b2_ref[...]
    pw = jnp.dot(posb_ref[...], w3b_ref[...], preferred_element_type=jnp.float32)
    c_ref[...] = jnp.dot(x, w3a_ref[...], preferred_element_type=jnp.float32) + pw
    e_ref[...] = pw[:, : _F]


def _conv2_body(gc_ref, e_ref, b3_ref, w4_ref, b4_ref, w5_ref, b5_ref, w6_ref,
                b6_ref, out_ref, acc_scr):
    i = pl.program_id(0)
    h1 = jnp.maximum(
        gc_ref[:, :, : _F] - e_ref[...][:, None, :] + b3_ref[...][None, :, :],
        0.0)
    h1 = h1.reshape(_RB * _K, _F)
    h2 = jnp.maximum(
        jnp.dot(h1, w4_ref[...], preferred_element_type=jnp.float32)
        + b4_ref[...], 0.0)
    h3 = jnp.dot(h2.astype(jnp.bfloat16), w5_ref[...],
                 preferred_element_type=jnp.float32)
    gblk = jnp.max(h3, axis=0, keepdims=True)

    @pl.when(i == 0)
    def _init():
        acc_scr[...] = gblk

    @pl.when(i > 0)
    def _acc():
        acc_scr[...] = jnp.maximum(acc_scr[...], gblk)

    @pl.when(i == _GRID - 1)
    def _fin():
        gfin = acc_scr[...] + b5_ref[...]
        out_ref[...] = jnp.maximum(
            jnp.dot(gfin, w6_ref[...], preferred_element_type=jnp.float32)
            + b6_ref[...], 0.0)


@functools.cache
def _gather_fn(ch):
    info = plsc.get_sparse_core_info()
    nc, ns = info.num_cores, info.num_subcores
    nw = nc * ns
    epad = nw * ch * _CW
    mesh = plsc.VectorSubcoreMesh(core_axis_name="c", subcore_axis_name="s")

    @functools.partial(
        pl.kernel,
        mesh=mesh,
        out_type=jax.ShapeDtypeStruct((epad, _GF), jnp.float32),
        scratch_types=[
            pltpu.VMEM((ch, _CW), jnp.int32),
            pltpu.VMEM((_CW, _GF), jnp.float32),
            pltpu.VMEM((_CW, _GF), jnp.float32),
            pltpu.SemaphoreType.DMA,
            pltpu.SemaphoreType.DMA,
        ],
    )
    def gather(table_hbm, idx_hbm, out_hbm, idx_v, rows_a, rows_b, sem_a,
               sem_b):
        wid = lax.axis_index("s") * nc + lax.axis_index("c")
        base = wid * ch * _CW
        pltpu.sync_copy(idx_hbm.at[wid], idx_v)
        pltpu.async_copy(table_hbm.at[idx_v.at[0]], rows_a, sem_a)

        def pair(p, carry):
            j = p * 2
            pltpu.async_copy(table_hbm.at[idx_v.at[j + 1]], rows_b, sem_b)
            pltpu.make_async_copy(table_hbm.at[idx_v.at[j]], rows_a,
                                  sem_a).wait()
            pltpu.sync_copy(rows_a, out_hbm.at[pl.ds(base + j * _CW, _CW)])

            @pl.when(p + 1 < ch // 2)
            def _next():
                pltpu.async_copy(table_hbm.at[idx_v.at[j + 2]], rows_a, sem_a)

            pltpu.make_async_copy(table_hbm.at[idx_v.at[j + 1]], rows_b,
                                  sem_b).wait()
            pltpu.sync_copy(rows_b, out_hbm.at[pl.ds(base + (j + 1) * _CW,
                                                     _CW)])
            return carry

        lax.fori_loop(0, ch // 2, pair, 0)

    return gather


def _sc_gather(table, idx3, ch):
    return _gather_fn(ch)(table, idx3)


def kernel(pos, batch, W1, b1, W2, b2, W3, b3, W4, b4, W5, b5, W6, b6):
    del batch
    info = plsc.get_sparse_core_info()
    nw = info.num_cores * info.num_subcores
    ch = -(-_E // (nw * _CW))
    ch += ch % 2
    epad = nw * ch * _CW

    post = jnp.pad(pos.T, ((0, 0), (0, _NP - _N)))
    wsum = jnp.pad(W1[:3] + W1[3:], ((0, 0), (0, _GF - _F)))
    wb = W1[3:]
    idx, u, v = pl.pallas_call(
        _topk_uv_body,
        grid=(_GRIDT,),
        in_specs=[
            pl.BlockSpec((_RBT, 3), lambda i: (i, 0)),
            pl.BlockSpec((3, _NP), lambda i: (0, 0)),
            pl.BlockSpec((3, _GF), lambda i: (0, 0)),
            pl.BlockSpec((3, _F), lambda i: (0, 0)),
        ],
        out_specs=[
            pl.BlockSpec((_RBT, _K), lambda i: (i, 0)),
            pl.BlockSpec((_RBT, _GF), lambda i: (i, 0)),
            pl.BlockSpec((_RBT, _F), lambda i: (i, 0)),
        ],
        out_shape=[
            jax.ShapeDtypeStruct((_N, _K), jnp.int32),
            jax.ShapeDtypeStruct((_N, _GF), jnp.float32),
            jax.ShapeDtypeStruct((_N, _F), jnp.float32),
        ],
        scratch_shapes=[pltpu.VMEM((_RBT, _NP), jnp.float32)],
    )(pos, post, wsum, wb)

    src = idx.reshape(-1)
    pad = (jnp.arange(epad - _E, dtype=jnp.int32) * 37) % _N
    src3 = jnp.concatenate([src, pad]).reshape(nw, ch, _CW)

    gu3 = _sc_gather(u, src3, ch).reshape(epad // _K, _K, _GF)

    b1r, b2r = b1.reshape(1, _F), b2.reshape(1, _F)
    w3a = jnp.pad(W3[:_F], ((0, 0), (0, _GF - _F)))
    w3b = jnp.pad(W3[_F:], ((0, 0), (0, _GF - _F)))
    c, e = pl.pallas_call(
        _conv1_body,
        grid=(_GRID,),
        in_specs=[
            pl.BlockSpec((_RB, _K, _GF), lambda i: (i, 0, 0)),
            pl.BlockSpec((_RB, _F), lambda i: (i, 0)),
            pl.BlockSpec((_RB, 3), lambda i: (i, 0)),
            pl.BlockSpec((1, _F), lambda i: (0, 0)),
            pl.BlockSpec((1, _F), lambda i: (0, 0)),
            pl.BlockSpec((_F, _F), lambda i: (0, 0)),
            pl.BlockSpec((_F, _GF), lambda i: (0, 0)),
            pl.BlockSpec((3, _GF), lambda i: (0, 0)),
        ],
        out_specs=[
            pl.BlockSpec((_RB, _GF), lambda i: (i, 0)),
            pl.BlockSpec((_RB, _F), lambda i: (i, 0)),
        ],
        out_shape=[
            jax.ShapeDtypeStruct((_N, _GF), jnp.float32),
            jax.ShapeDtypeStruct((_N, _F), jnp.float32),
        ],
    )(gu3, v, pos, b1r, b2r, W2, w3a, w3b)

    gc3 = _sc_gather(c, src3, ch).reshape(epad // _K, _K, _GF)

    out = pl.pallas_call(
        _conv2_body,
        grid=(_GRID,),
        in_specs=[
            pl.BlockSpec((_RB, _K, _GF), lambda i: (i, 0, 0)),
            pl.BlockSpec((_RB, _F), lambda i: (i, 0)),
            pl.BlockSpec((1, _F), lambda i: (0, 0)),
            pl.BlockSpec((_F, 128), lambda i: (0, 0)),
            pl.BlockSpec((1, 128), lambda i: (0, 0)),
            pl.BlockSpec((128, 1024), lambda i: (0, 0)),
            pl.BlockSpec((1, 1024), lambda i: (0, 0)),
            pl.BlockSpec((1024, 256), lambda i: (0, 0)),
            pl.BlockSpec((1, 256), lambda i: (0, 0)),
        ],
        out_specs=pl.BlockSpec((1, 256), lambda i: (0, 0)),
        out_shape=jax.ShapeDtypeStruct((1, 256), jnp.float32),
        scratch_shapes=[pltpu.VMEM((1, 1024), jnp.float32)],
    )(gc3, e, b3.reshape(1, _F), W4, b4.reshape(1, 128),
      W5.astype(jnp.bfloat16), b5.reshape(1, 1024), W6, b6.reshape(1, 256))
    return out
